# baseline (speedup 1.0000x reference)
import jax
import jax.numpy as jnp
from jax.experimental import pallas as pl
from jax.experimental.pallas import tpu as pltpu

_K = 6
_RC = 8


def _scatter_kernel(x_hbm, mask_ref, o_hbm, cap8, inbuf, outbuf,
                    in_sem, out_sem):
    B, V = x_hbm.shape
    nchunks = B // _RC
    n_forbidden = jnp.sum(mask_ref[...].astype(jnp.int32))

    @pl.when(n_forbidden > 0)
    def _apply_scatter():
        m8 = jnp.broadcast_to(mask_ref[0:1, :] != 0, (_RC, V))
        cap8[...] = jnp.where(m8, -jnp.inf, jnp.inf).astype(cap8.dtype)

        def in_copy(c, s):
            rows = pl.ds(c * _RC, _RC)
            return pltpu.make_async_copy(x_hbm.at[rows, :], inbuf.at[s],
                                         in_sem.at[s])

        def out_copy(c, s):
            rows = pl.ds(c * _RC, _RC)
            return pltpu.make_async_copy(outbuf.at[s], o_hbm.at[rows, :],
                                         out_sem.at[s])

        for c in range(min(_K, nchunks)):
            in_copy(c, c).start()

        for c in range(nchunks):
            s = c % _K
            in_copy(c, s).wait()
            if c >= _K:
                out_copy(c - _K, s).wait()
            outbuf[s] = jnp.minimum(inbuf[s], cap8[...])
            out_copy(c, s).start()
            nxt = c + _K
            if nxt < nchunks:
                in_copy(nxt, s).start()

        for c in range(max(0, nchunks - _K), nchunks):
            out_copy(c, c % _K).wait()


def kernel(logits, generated_so_far, forbidden_token_mask):
    del generated_so_far
    B, V = logits.shape
    mask2d = forbidden_token_mask.astype(jnp.int8).reshape(1, V)
    return pl.pallas_call(
        _scatter_kernel,
        in_specs=[
            pl.BlockSpec(memory_space=pltpu.MemorySpace.HBM),
            pl.BlockSpec(memory_space=pltpu.MemorySpace.VMEM),
        ],
        out_specs=pl.BlockSpec(memory_space=pltpu.MemorySpace.HBM),
        out_shape=jax.ShapeDtypeStruct((B, V), logits.dtype),
        input_output_aliases={0: 0},
        scratch_shapes=[
            pltpu.VMEM((_RC, V), logits.dtype),
            pltpu.VMEM((_K, _RC, V), logits.dtype),
            pltpu.VMEM((_K, _RC, V), logits.dtype),
            pltpu.SemaphoreType.DMA((_K,)),
            pltpu.SemaphoreType.DMA((_K,)),
        ],
    )(logits, mask2d)
